# Initial kernel scaffold; baseline (speedup 1.0000x reference)
import jax
import jax.numpy as jnp
from jax.experimental import pallas as pl


def kernel(z_e, codebook):
    raise NotImplementedError("write your pallas kernel here")


# speedup vs baseline: 1.0672x; 1.0672x over previous
import jax
import jax.numpy as jnp
from jax.experimental import pallas as pl
from jax.experimental.pallas import tpu as pltpu

M = 16384
K = 8192
D = 256
MT = 512
CHUNK = 2736
CPAD = 2816
NCHUNK = 3
LANES = 128
KBIG = 1 << 24


def _vq_argmin_kernel(z_ref, insq_ref, cb_ref, csq_ref, out_ref, accv_sc, acci_sc):
    j = pl.program_id(1)

    z = z_ref[...]
    cb = cb_ref[...]

    m = jax.lax.dot_general(
        z, cb,
        dimension_numbers=(((1,), (1,)), ((), ())),
        preferred_element_type=jnp.float32,
    )
    dist = (csq_ref[...] + insq_ref[...]) - 2.0 * m

    lane = jax.lax.broadcasted_iota(jnp.int32, (MT, LANES), 1)
    mn = jnp.full((MT, LANES), jnp.inf, dtype=jnp.float32)
    ix = jnp.zeros((MT, LANES), dtype=jnp.int32)
    base = j * CHUNK
    for t in range(CPAD // LANES):
        d = dist[:, t * LANES:(t + 1) * LANES]
        better = d < mn
        mn = jnp.where(better, d, mn)
        ix = jnp.where(better, lane + (t * LANES), ix)
    ix = ix + base

    rm = jnp.min(mn, axis=1, keepdims=True)
    ci = jnp.min(jnp.where(mn == rm, ix, KBIG), axis=1, keepdims=True)

    @pl.when(j == 0)
    def _init():
        accv_sc[...] = rm.astype(jnp.bfloat16).astype(jnp.float32)
        acci_sc[...] = ci

    @pl.when(j > 0)
    def _merge():
        av = accv_sc[...]
        ai = acci_sc[...]
        take = (rm < av) | ((rm == av) & (ci < ai))
        accv_sc[...] = jnp.where(take, rm, av).astype(jnp.bfloat16).astype(jnp.float32)
        acci_sc[...] = jnp.where(take, ci, ai)

    @pl.when(j == NCHUNK - 1)
    def _emit():
        out_ref[...] = acci_sc[...]


def kernel(z_e, codebook):
    z_flat = z_e.reshape(-1, D)
    in_sqr = jnp.sum(z_flat ** 2, axis=1, keepdims=True)
    cb_sqr = jnp.sum(codebook ** 2, axis=1)

    pad = NCHUNK * CPAD - K - (CPAD - CHUNK) * (NCHUNK - 1)
    cb_chunks, csq_chunks = [], []
    for c in range(NCHUNK):
        lo = c * CHUNK
        hi = min(lo + CHUNK, K)
        cb_chunks.append(jnp.pad(codebook[lo:hi], ((0, CPAD - (hi - lo)), (0, 0))))
        csq_chunks.append(jnp.pad(cb_sqr[lo:hi], (0, CPAD - (hi - lo)),
                                  constant_values=jnp.inf))
    cb_pad = jnp.concatenate(cb_chunks, axis=0)
    csq_pad = jnp.concatenate(csq_chunks)[None, :]

    out = pl.pallas_call(
        _vq_argmin_kernel,
        grid=(M // MT, NCHUNK),
        in_specs=[
            pl.BlockSpec((MT, D), lambda i, j: (i, 0)),
            pl.BlockSpec((MT, 1), lambda i, j: (i, 0)),
            pl.BlockSpec((CPAD, D), lambda i, j: (j, 0)),
            pl.BlockSpec((1, CPAD), lambda i, j: (0, j)),
        ],
        out_specs=pl.BlockSpec((MT, 1), lambda i, j: (i, 0)),
        out_shape=jax.ShapeDtypeStruct((M, 1), jnp.int32),
        scratch_shapes=[
            pltpu.VMEM((MT, 1), jnp.float32),
            pltpu.VMEM((MT, 1), jnp.int32),
        ],
        compiler_params=pltpu.CompilerParams(
            dimension_semantics=("parallel", "arbitrary"),
        ),
    )(z_flat, in_sqr, cb_pad, csq_pad)

    return out.reshape(z_e.shape[:-1])


# speedup vs baseline: 1.1536x; 1.0810x over previous
import jax
import jax.numpy as jnp
from jax.experimental import pallas as pl
from jax.experimental.pallas import tpu as pltpu

M = 16384
K = 8192
D = 256
MT = 1024
CHUNK = 2736
CPAD = 2816
NCHUNK = 3
LANES = 128
KBIG = 1 << 24


def _vq_argmin_kernel(z_ref, insq_ref, cb_ref, csq_ref, out_ref, accv_sc, acci_sc):
    j = pl.program_id(1)

    z = z_ref[...]
    cb = cb_ref[...]

    m = jax.lax.dot_general(
        z, cb,
        dimension_numbers=(((1,), (1,)), ((), ())),
        preferred_element_type=jnp.float32,
    )
    dist = (csq_ref[...] + insq_ref[...]) - 2.0 * m

    lane = jax.lax.broadcasted_iota(jnp.int32, (MT, LANES), 1)
    mn = jnp.full((MT, LANES), jnp.inf, dtype=jnp.float32)
    ix = jnp.zeros((MT, LANES), dtype=jnp.int32)
    base = j * CHUNK
    for t in range(CPAD // LANES):
        d = dist[:, t * LANES:(t + 1) * LANES]
        better = d < mn
        mn = jnp.where(better, d, mn)
        ix = jnp.where(better, lane + (t * LANES), ix)
    ix = ix + base

    rm = jnp.min(mn, axis=1, keepdims=True)
    ci = jnp.min(jnp.where(mn == rm, ix, KBIG), axis=1, keepdims=True)

    @pl.when(j == 0)
    def _init():
        accv_sc[...] = rm.astype(jnp.bfloat16).astype(jnp.float32)
        acci_sc[...] = ci

    @pl.when(j > 0)
    def _merge():
        av = accv_sc[...]
        ai = acci_sc[...]
        take = (rm < av) | ((rm == av) & (ci < ai))
        accv_sc[...] = jnp.where(take, rm, av).astype(jnp.bfloat16).astype(jnp.float32)
        acci_sc[...] = jnp.where(take, ci, ai)

    @pl.when(j == NCHUNK - 1)
    def _emit():
        out_ref[...] = acci_sc[...]


def kernel(z_e, codebook):
    z_flat = z_e.reshape(-1, D)
    in_sqr = jnp.sum(z_flat ** 2, axis=1, keepdims=True)
    cb_sqr = jnp.sum(codebook ** 2, axis=1)

    pad = NCHUNK * CPAD - K - (CPAD - CHUNK) * (NCHUNK - 1)
    cb_chunks, csq_chunks = [], []
    for c in range(NCHUNK):
        lo = c * CHUNK
        hi = min(lo + CHUNK, K)
        cb_chunks.append(jnp.pad(codebook[lo:hi], ((0, CPAD - (hi - lo)), (0, 0))))
        csq_chunks.append(jnp.pad(cb_sqr[lo:hi], (0, CPAD - (hi - lo)),
                                  constant_values=jnp.inf))
    cb_pad = jnp.concatenate(cb_chunks, axis=0)
    csq_pad = jnp.concatenate(csq_chunks)[None, :]

    out = pl.pallas_call(
        _vq_argmin_kernel,
        grid=(M // MT, NCHUNK),
        in_specs=[
            pl.BlockSpec((MT, D), lambda i, j: (i, 0)),
            pl.BlockSpec((MT, 1), lambda i, j: (i, 0)),
            pl.BlockSpec((CPAD, D), lambda i, j: (j, 0)),
            pl.BlockSpec((1, CPAD), lambda i, j: (0, j)),
        ],
        out_specs=pl.BlockSpec((MT, 1), lambda i, j: (i, 0)),
        out_shape=jax.ShapeDtypeStruct((M, 1), jnp.int32),
        scratch_shapes=[
            pltpu.VMEM((MT, 1), jnp.float32),
            pltpu.VMEM((MT, 1), jnp.int32),
        ],
        compiler_params=pltpu.CompilerParams(
            dimension_semantics=("parallel", "arbitrary"),
        ),
    )(z_flat, in_sqr, cb_pad, csq_pad)

    return out.reshape(z_e.shape[:-1])
